# baseline (speedup 1.0000x reference)
import functools

import jax
import jax.numpy as jnp
import numpy as np
from jax import lax
from jax.experimental import pallas as pl
from jax.experimental.pallas import tpu as pltpu
from jax.experimental.pallas import tpu_sc as plsc

_B = 128
_S = 8192
_NSUP = 32
_ROWS_PER_W = 4
_NEG = float("-inf")

_SEL_PACKED = (
    "7A71>4CG9C9@E;:>18>?>51G==:A4@1A5ECC79>>;15DD35C27??A2FD?5@41C2>=95G@DC"
    "727G@C779A@4>FD5=4D5DD@54773:31G:@@759CGG81@=8A@@792:21EG"
)


def _sel_packed_const() -> np.ndarray:
    return np.frombuffer(_SEL_PACKED.encode(), np.uint8).astype(np.int32) - 48


def _tree_max16(load):
    vs = [load(c) for c in range(16)]
    while len(vs) > 1:
        vs = [jnp.maximum(vs[i], vs[i + 1]) for i in range(0, len(vs), 2)]
    return vs[0]


def _sc_body(int_hbm, x_hbm, sel_hbm, mx_hbm, tok_hbm, pos_hbm,
             ibuf, xbuf, sel_v, tok4_v, res4_v,
             sem_i, sem_x, sem_o):
    wid = lax.axis_index("s") * 2 + lax.axis_index("c")
    iota16 = jnp.arange(16, dtype=jnp.int32)
    row0 = wid * _ROWS_PER_W

    for r in range(_ROWS_PER_W):
        pltpu.async_copy(int_hbm.at[row0 + r, 0],
                         ibuf.at[pl.ds(r * _S, _S)], sem_i.at[r])
        pltpu.async_copy(x_hbm.at[row0 + r],
                         xbuf.at[pl.ds(r * _S, _S)], sem_x.at[r])
    pltpu.sync_copy(sel_hbm, sel_v)

    def row_body(r, _):
        row = row0 + r
        rbase = r * _S
        pltpu.make_async_copy(int_hbm.at[row, 0],
                              ibuf.at[pl.ds(rbase, _S)], sem_i.at[r]).wait()

        def super_body(s, carry):
            s0a, s1a = carry
            mx = jnp.max(_tree_max16(
                lambda c: ibuf[pl.ds(rbase + s * 256 + c * 16, 16)]))
            mv = jnp.full((16,), mx)
            s0a = jnp.where(iota16 == s, mv, s0a)
            s1a = jnp.where(iota16 == s - 16, mv, s1a)
            return (s0a, s1a)

        s0, s1 = lax.fori_loop(
            0, _NSUP, super_body,
            (jnp.full((16,), _NEG, jnp.float32),
             jnp.full((16,), _NEG, jnp.float32)))

        def extract_body(k, carry):
            s0, s1, t5 = carry
            gmax = jnp.max(jnp.maximum(s0, s1))
            gv = jnp.full((16,), gmax)
            f0 = plsc.all_reduce_ffs(s0 == gv)
            f1 = plsc.all_reduce_ffs(s1 == gv)
            fs = jnp.min(jnp.where(f0 < 16, f0, 16 + f1))

            def fc_body(c, carry2):
                fc_v, fl_v = carry2
                eqm = ibuf[pl.ds(rbase + fs * 256 + c * 16, 16)] == gv
                l = plsc.all_reduce_ffs(eqm)
                take = (fc_v == 999) & (l < 16)
                return (jnp.where(take, c, fc_v), jnp.where(take, l, fl_v))

            fc_v, fl_v = lax.fori_loop(
                0, 16, fc_body,
                (jnp.full((16,), 999, jnp.int32),
                 jnp.full((16,), 0, jnp.int32)))
            fc = jnp.min(fc_v)
            fl = jnp.min(fl_v)

            base = fs * 256 + fc * 16
            t5 = jnp.where(iota16 == k, base + fl, t5)

            v = ibuf[pl.ds(rbase + base, 16)]
            ibuf[pl.ds(rbase + base, 16)] = jnp.where(iota16 == fl, _NEG, v)
            mx = jnp.max(_tree_max16(
                lambda c: ibuf[pl.ds(rbase + fs * 256 + c * 16, 16)]))
            mv = jnp.full((16,), mx)
            s0 = jnp.where(iota16 == fs, mv, s0)
            s1 = jnp.where(iota16 == fs - 16, mv, s1)
            return (s0, s1, t5)

        _, _, t5 = lax.fori_loop(0, 5, extract_body,
                                 (s0, s1, jnp.zeros((16,), jnp.int32)))

        def _gather16(vec, idx):
            return lax.gather(
                vec, idx[:, None],
                lax.GatherDimensionNumbers(
                    offset_dims=(), collapsed_slice_dims=(0,),
                    start_index_map=(0,)),
                (1,), mode=lax.GatherScatterMode.PROMISE_IN_BOUNDS)

        chunk = sel_v[pl.ds(row & ~15, 16)]
        val = _gather16(chunk, jnp.full((16,), row & 15, jnp.int32))
        a = (val * 13) >> 6
        b = val - 5 * a
        idx = jnp.where(iota16 == 0, a, jnp.where(iota16 == 1, b, 0))
        lane01 = iota16 < 2
        mp = jnp.where(lane01, _gather16(t5, idx), 0)
        res4_v[pl.ds(r * 16, 16)] = mp

        pltpu.make_async_copy(x_hbm.at[row],
                              xbuf.at[pl.ds(rbase, _S)], sem_x.at[r]).wait()
        gathered = plsc.load_gather(xbuf, [rbase + mp], mask=lane01)
        tok4_v[pl.ds(r * 16, 16)] = jnp.where(
            lane01, gathered.astype(jnp.int32), 0)
        plsc.store_scatter(xbuf, [rbase + mp],
                           jnp.full((16,), 1.0, jnp.float32), mask=lane01)
        pltpu.async_copy(xbuf.at[pl.ds(rbase, _S)], mx_hbm.at[row],
                         sem_o.at[r])
        return 0

    lax.fori_loop(0, _ROWS_PER_W, row_body, 0)

    pltpu.sync_copy(res4_v, pos_hbm.at[pl.ds(row0 * 16, _ROWS_PER_W * 16)])
    pltpu.sync_copy(tok4_v, tok_hbm.at[pl.ds(row0 * 16, _ROWS_PER_W * 16)])

    def drain_body(r, _):
        pltpu.make_async_copy(xbuf.at[pl.ds(r * _S, _S)],
                              mx_hbm.at[row0 + r], sem_o.at[r]).wait()
        return 0

    lax.fori_loop(0, _ROWS_PER_W, drain_body, 0)


_sc_fused = functools.partial(
    pl.kernel,
    mesh=plsc.VectorSubcoreMesh(core_axis_name="c", subcore_axis_name="s"),
    compiler_params=pltpu.CompilerParams(needs_layout_passes=False),
    out_type=[
        jax.ShapeDtypeStruct((_B, _S), jnp.float32),
        jax.ShapeDtypeStruct((_B * 16,), jnp.int32),
        jax.ShapeDtypeStruct((_B * 16,), jnp.int32),
    ],
    scratch_types=[
        pltpu.VMEM((_ROWS_PER_W * _S,), jnp.float32),
        pltpu.VMEM((_ROWS_PER_W * _S,), jnp.float32),
        pltpu.VMEM((_B,), jnp.int32),
        pltpu.VMEM((_ROWS_PER_W * 16,), jnp.int32),
        pltpu.VMEM((_ROWS_PER_W * 16,), jnp.int32),
        pltpu.SemaphoreType.DMA((_ROWS_PER_W,)),
        pltpu.SemaphoreType.DMA((_ROWS_PER_W,)),
        pltpu.SemaphoreType.DMA((_ROWS_PER_W,)),
    ],
)(_sc_body)


def kernel(x, intensity_):
    selp = jnp.asarray(_sel_packed_const())
    mask_x, tok_flat, pos_flat = _sc_fused(intensity_, x, selp)
    tok16 = tok_flat.reshape(_B, 16)
    pos16 = pos_flat.reshape(_B, 16)
    return (mask_x, tok16[:, :8], pos16[:, :8])


# speedup vs baseline: 1.0485x; 1.0485x over previous
import functools

import jax
import jax.numpy as jnp
import numpy as np
from jax import lax
from jax.experimental import pallas as pl
from jax.experimental.pallas import tpu as pltpu
from jax.experimental.pallas import tpu_sc as plsc

_B = 128
_S = 8192
_NSUP = 32
_ROWS_PER_W = 4
_NEG = float("-inf")

_SEL_PACKED = (
    "7A71>4CG9C9@E;:>18>?>51G==:A4@1A5ECC79>>;15DD35C27??A2FD?5@41C2>=95G@DC"
    "727G@C779A@4>FD5=4D5DD@54773:31G:@@759CGG81@=8A@@792:21EG"
)


def _sel16_const() -> np.ndarray:
    v = np.frombuffer(_SEL_PACKED.encode(), np.uint8).astype(np.int32) - 48
    out = np.zeros((_B, 16), np.int32)
    out[:, 0] = v // 5
    out[:, 1] = v % 5
    return out


def _tree_max16(load):
    vs = [load(c) for c in range(16)]
    while len(vs) > 1:
        vs = [jnp.maximum(vs[i], vs[i + 1]) for i in range(0, len(vs), 2)]
    return vs[0]


def _sc_body(int_hbm, x_hbm, sel_hbm, mx_hbm, tok_hbm, pos_hbm,
             ibuf, xbuf, sel4_v, tok4_v, res4_v,
             sem_i, sem_x, sem_o):
    wid = lax.axis_index("s") * 2 + lax.axis_index("c")
    iota16 = jnp.arange(16, dtype=jnp.int32)
    row0 = wid * _ROWS_PER_W

    for r in range(_ROWS_PER_W):
        pltpu.async_copy(int_hbm.at[row0 + r, 0],
                         ibuf.at[pl.ds(r * _S, _S)], sem_i.at[r])
        pltpu.async_copy(x_hbm.at[row0 + r],
                         xbuf.at[pl.ds(r * _S, _S)], sem_x.at[r])
    pltpu.sync_copy(sel_hbm.at[pl.ds(row0 * 16, _ROWS_PER_W * 16)], sel4_v)

    def row_body(r, _):
        row = row0 + r
        rbase = r * _S
        pltpu.make_async_copy(int_hbm.at[row, 0],
                              ibuf.at[pl.ds(rbase, _S)], sem_i.at[r]).wait()

        def super_body(s, carry):
            s0a, s1a = carry
            mx = jnp.max(_tree_max16(
                lambda c: ibuf[pl.ds(rbase + s * 256 + c * 16, 16)]))
            mv = jnp.full((16,), mx)
            s0a = jnp.where(iota16 == s, mv, s0a)
            s1a = jnp.where(iota16 == s - 16, mv, s1a)
            return (s0a, s1a)

        s0, s1 = lax.fori_loop(
            0, _NSUP, super_body,
            (jnp.full((16,), _NEG, jnp.float32),
             jnp.full((16,), _NEG, jnp.float32)))

        def extract_body(k, carry):
            s0, s1, t5 = carry
            gmax = jnp.max(jnp.maximum(s0, s1))
            gv = jnp.full((16,), gmax)
            f0 = plsc.all_reduce_ffs(s0 == gv)
            f1 = plsc.all_reduce_ffs(s1 == gv)
            fs = jnp.min(jnp.where(f0 < 16, f0, 16 + f1))

            vs = [ibuf[pl.ds(rbase + fs * 256 + c * 16, 16)]
                  for c in range(16)]
            fc_v = jnp.full((16,), 999, jnp.int32)
            fl_v = jnp.full((16,), 0, jnp.int32)
            for c in range(16):
                l = plsc.all_reduce_ffs(vs[c] == gv)
                take = (fc_v == 999) & (l < 16)
                fc_v = jnp.where(take, c, fc_v)
                fl_v = jnp.where(take, l, fl_v)
            fc = jnp.min(fc_v)
            fl = jnp.min(fl_v)

            base = fs * 256 + fc * 16
            t5 = jnp.where(iota16 == k, base + fl, t5)

            knocked = (iota16 == fl_v)
            negv = jnp.full((16,), _NEG, jnp.float32)
            vrep = [jnp.where((fc_v == c) & knocked, negv, vs[c])
                    for c in range(16)]
            ibuf[pl.ds(rbase + base, 16)] = jnp.where(
                knocked, negv, ibuf[pl.ds(rbase + base, 16)])
            while len(vrep) > 1:
                vrep = [jnp.maximum(vrep[i], vrep[i + 1])
                        for i in range(0, len(vrep), 2)]
            mx = jnp.max(vrep[0])
            mv = jnp.full((16,), mx)
            s0 = jnp.where(iota16 == fs, mv, s0)
            s1 = jnp.where(iota16 == fs - 16, mv, s1)
            return (s0, s1, t5)

        _, _, t5 = lax.fori_loop(0, 5, extract_body,
                                 (s0, s1, jnp.zeros((16,), jnp.int32)))

        sv = sel4_v[pl.ds(r * 16, 16)]
        lane01 = iota16 < 2
        gathered_pos = lax.gather(
            t5, sv[:, None],
            lax.GatherDimensionNumbers(
                offset_dims=(), collapsed_slice_dims=(0,),
                start_index_map=(0,)),
            (1,), mode=lax.GatherScatterMode.PROMISE_IN_BOUNDS)
        mp = jnp.where(lane01, gathered_pos, 0)
        res4_v[pl.ds(r * 16, 16)] = mp

        pltpu.make_async_copy(x_hbm.at[row],
                              xbuf.at[pl.ds(rbase, _S)], sem_x.at[r]).wait()
        gathered = plsc.load_gather(xbuf, [rbase + mp], mask=lane01)
        tok4_v[pl.ds(r * 16, 16)] = jnp.where(
            lane01, gathered.astype(jnp.int32), 0)
        plsc.store_scatter(xbuf, [rbase + mp],
                           jnp.full((16,), 1.0, jnp.float32), mask=lane01)
        pltpu.async_copy(xbuf.at[pl.ds(rbase, _S)], mx_hbm.at[row],
                         sem_o.at[r])
        return 0

    lax.fori_loop(0, _ROWS_PER_W, row_body, 0)

    pltpu.sync_copy(res4_v, pos_hbm.at[pl.ds(row0 * 16, _ROWS_PER_W * 16)])
    pltpu.sync_copy(tok4_v, tok_hbm.at[pl.ds(row0 * 16, _ROWS_PER_W * 16)])

    def drain_body(r, _):
        pltpu.make_async_copy(xbuf.at[pl.ds(r * _S, _S)],
                              mx_hbm.at[row0 + r], sem_o.at[r]).wait()
        return 0

    lax.fori_loop(0, _ROWS_PER_W, drain_body, 0)


_sc_fused = functools.partial(
    pl.kernel,
    mesh=plsc.VectorSubcoreMesh(core_axis_name="c", subcore_axis_name="s"),
    compiler_params=pltpu.CompilerParams(needs_layout_passes=False),
    out_type=[
        jax.ShapeDtypeStruct((_B, _S), jnp.float32),
        jax.ShapeDtypeStruct((_B * 16,), jnp.int32),
        jax.ShapeDtypeStruct((_B * 16,), jnp.int32),
    ],
    scratch_types=[
        pltpu.VMEM((_ROWS_PER_W * _S,), jnp.float32),
        pltpu.VMEM((_ROWS_PER_W * _S,), jnp.float32),
        pltpu.VMEM((_ROWS_PER_W * 16,), jnp.int32),
        pltpu.VMEM((_ROWS_PER_W * 16,), jnp.int32),
        pltpu.VMEM((_ROWS_PER_W * 16,), jnp.int32),
        pltpu.SemaphoreType.DMA((_ROWS_PER_W,)),
        pltpu.SemaphoreType.DMA((_ROWS_PER_W,)),
        pltpu.SemaphoreType.DMA((_ROWS_PER_W,)),
    ],
)(_sc_body)


def kernel(x, intensity_):
    sel16 = jnp.asarray(_sel16_const()).reshape(_B * 16)
    mask_x, tok_flat, pos_flat = _sc_fused(intensity_, x, sel16)
    tok16 = tok_flat.reshape(_B, 16)
    pos16 = pos_flat.reshape(_B, 16)
    return (mask_x, tok16[:, :8], pos16[:, :8])
